# speedup vs baseline: 1.5870x; 1.5870x over previous
import functools

import jax
import jax.numpy as jnp
from jax.experimental import pallas as pl
from jax.experimental.pallas import tpu as pltpu

_BR = 16


def _jsd_body(x_ref, tgt_ref, out_ref):
    xb = x_ref[...]
    tgt = tgt_ref[...]
    col = jax.lax.broadcasted_iota(jnp.int32, xb.shape, 1)
    is_tgt = col == tgt
    t = jnp.log(jnp.exp(xb) * 0.5) - xb
    m1 = jnp.min(jnp.where(is_tgt, jnp.inf, t), axis=1, keepdims=True)
    xt = jnp.sum(jnp.where(is_tgt, xb, 0.0), axis=1, keepdims=True)
    d2 = jnp.log((jnp.exp(xt) + 1.0) * 0.5)
    out_ref[...] = -(m1 + d2)


def kernel(x, target):
    Bn, Cn = x.shape
    tgt = target.astype(jnp.int32).reshape(Bn, 1)
    grid = (Bn // _BR,)
    return pl.pallas_call(
        _jsd_body,
        grid=grid,
        in_specs=[
            pl.BlockSpec((_BR, Cn), lambda i: (i, 0)),
            pl.BlockSpec((_BR, 1), lambda i: (i, 0)),
        ],
        out_specs=pl.BlockSpec((_BR, 1), lambda i: (i, 0)),
        out_shape=jax.ShapeDtypeStruct((Bn, 1), x.dtype),
    )(x, tgt)
